# baseline (speedup 1.0000x reference)
import functools

import jax
import jax.numpy as jnp
from jax import lax
from jax.experimental import pallas as pl
from jax.experimental.pallas import tpu as pltpu
from jax.experimental.pallas import tpu_sc as plsc

B = 1024
D = 128
N_EP = 100000
K = 8
CBLK = 2048
NBLK = (N_EP + CBLK - 1) // CBLK
NPAD = NBLK * CBLK
NCAND = NBLK * K
CPAD = 512
NEG = -1e30
BIGF = 1e9


def _topk_body(query_ref, wq_ref, keys_ref, cv_ref, ci_ref, qn_ref, thr_ref):
    i = pl.program_id(0)

    @pl.when(i == 0)
    def _init():
        q = lax.dot_general(query_ref[...], wq_ref[...],
                            (((1,), (1,)), ((), ())),
                            preferred_element_type=jnp.float32)
        n = jnp.sqrt(jnp.sum(q * q, axis=1, keepdims=True))
        qn_ref[...] = q / jnp.maximum(n, 1e-12)
        thr_ref[...] = jnp.full((B,), NEG, jnp.float32)

    keys = keys_ref[...]
    kn = jnp.sqrt(jnp.sum(keys * keys, axis=1, keepdims=True))
    keysn = keys / jnp.maximum(kn, 1e-12)
    sim = lax.dot_general(qn_ref[...], keysn,
                          (((1,), (1,)), ((), ())),
                          preferred_element_type=jnp.float32)
    base = i * CBLK
    limitf = (N_EP - base).astype(jnp.float32)
    rowi = lax.broadcasted_iota(jnp.int32, (1, CBLK), 1).astype(jnp.float32)
    bias = jnp.where(rowi < limitf, 0.0, NEG)
    sim = sim + bias
    lcolf = lax.broadcasted_iota(jnp.int32, (B, CBLK), 1).astype(jnp.float32)
    basef = base.astype(jnp.float32)
    laneK = lax.broadcasted_iota(jnp.int32, (B, K), 1)

    thr = thr_ref[...]
    cnt = jnp.sum((sim >= thr[:, None]).astype(jnp.float32), axis=1)
    need = jnp.minimum(jnp.max(cnt), float(K)).astype(jnp.int32)

    def ext(t, carry):
        sim, cv, ci = carry
        m = jnp.max(sim, axis=1)
        lif = jnp.min(jnp.where(sim == m[:, None], lcolf, BIGF), axis=1)
        cv = jnp.where(laneK == t, m[:, None], cv)
        ci = jnp.where(laneK == t, lif[:, None] + basef, ci)
        sim = jnp.where(lcolf == lif[:, None], NEG, sim)
        return sim, cv, ci

    cv0 = jnp.full((B, K), NEG, jnp.float32)
    ci0 = jnp.zeros((B, K), jnp.float32)
    _, cv, ci = lax.fori_loop(0, need, ext, (sim, cv0, ci0))
    cv_ref[0] = cv
    ci_ref[0] = ci
    thr_ref[...] = jnp.maximum(thr, cv[:, K - 1])


def _run_topk(query, Wq, keys_pad):
    return pl.pallas_call(
        _topk_body,
        grid=(NBLK,),
        in_specs=[
            pl.BlockSpec((B, D), lambda i: (0, 0)),
            pl.BlockSpec((D, D), lambda i: (0, 0)),
            pl.BlockSpec((CBLK, D), lambda i: (i, 0)),
        ],
        out_specs=[
            pl.BlockSpec((1, B, K), lambda i: (i, 0, 0)),
            pl.BlockSpec((1, B, K), lambda i: (i, 0, 0)),
        ],
        out_shape=[
            jax.ShapeDtypeStruct((NBLK, B, K), jnp.float32),
            jax.ShapeDtypeStruct((NBLK, B, K), jnp.float32),
        ],
        scratch_shapes=[pltpu.VMEM((B, D), jnp.float32),
                        pltpu.VMEM((B,), jnp.float32)],
    )(query, Wq, keys_pad)


def _select_body(cv_ref, ci_ref, tv_ref, ti_ref):
    v = cv_ref[...]
    ci = ci_ref[...]
    pos = lax.broadcasted_iota(jnp.int32, (B, CPAD), 1).astype(jnp.float32)
    new_v, new_i = [], []
    for _ in range(K):
        m = jnp.max(v, axis=1)
        p = jnp.min(jnp.where(v == m[:, None], pos, BIGF), axis=1)
        hit = pos == p[:, None]
        new_v.append(m[:, None])
        new_i.append(jnp.max(jnp.where(hit, ci, -1.0), axis=1)[:, None])
        v = jnp.where(hit, NEG, v)
    tv_ref[...] = jnp.concatenate(new_v, axis=1)
    ti_ref[...] = jnp.concatenate(new_i, axis=1).astype(jnp.int32)


def _run_select(cvp, cip):
    return pl.pallas_call(
        _select_body,
        out_shape=[
            jax.ShapeDtypeStruct((B, K), jnp.float32),
            jax.ShapeDtypeStruct((B, K), jnp.int32),
        ],
    )(cvp, cip)


_NTOT = B * K
_CH = 128


def _make_sc_gather():
    info = plsc.get_sparse_core_info()
    nc, ns = info.num_cores, info.num_subcores
    nw = nc * ns
    bpw = _NTOT // nw
    nch = bpw // _CH
    mesh = plsc.VectorSubcoreMesh(core_axis_name="c", subcore_axis_name="s")

    @functools.partial(
        pl.kernel, mesh=mesh,
        out_type=jax.ShapeDtypeStruct((_NTOT, D), jnp.float32),
        scratch_types=[
            pltpu.VMEM((nch, _CH), jnp.int32),
            pltpu.VMEM((bpw, D), jnp.float32),
            pltpu.SemaphoreType.DMA,
        ],
    )
    def gather_k(values_hbm, idx_hbm, out_hbm, idx_v, rows_v, sem):
        wid = lax.axis_index("s") * nc + lax.axis_index("c")
        pltpu.sync_copy(idx_hbm.at[pl.ds(wid * nch, nch)], idx_v)
        cps = [
            pltpu.async_copy(values_hbm.at[idx_v.at[j]],
                             rows_v.at[pl.ds(j * _CH, _CH)], sem)
            for j in range(nch)
        ]
        for cp in cps:
            cp.wait()
        pltpu.sync_copy(rows_v, out_hbm.at[pl.ds(wid * bpw, bpw)])

    return gather_k


def _tail_body(query_ref, tv_ref, wv_ref, wg1_ref, wg2_ref, bg_ref,
               *sel_and_out):
    sel_refs = sel_and_out[:K]
    out_ref = sel_and_out[K]
    z = tv_ref[...] * 5.0
    m = jnp.max(z, axis=1, keepdims=True)
    e = jnp.exp(z - m)
    w = e / jnp.sum(e, axis=1, keepdims=True)
    r = w[:, 0:1] * sel_refs[0][...]
    for j in range(1, K):
        r = r + w[:, j:j + 1] * sel_refs[j][...]
    proj = lax.dot_general(r, wv_ref[...], (((1,), (1,)), ((), ())),
                           preferred_element_type=jnp.float32)
    g = (lax.dot_general(query_ref[...], wg1_ref[...],
                         (((1,), (1,)), ((), ())),
                         preferred_element_type=jnp.float32)
         + lax.dot_general(proj, wg2_ref[...], (((1,), (1,)), ((), ())),
                           preferred_element_type=jnp.float32)
         + bg_ref[...])
    gate = jax.nn.sigmoid(g)
    out_ref[...] = gate * proj


def _run_tail(query, tv, Wv, Wg1, Wg2, bg2d, sels):
    return pl.pallas_call(
        _tail_body,
        out_shape=jax.ShapeDtypeStruct((B, D), jnp.float32),
    )(query, tv, Wv, Wg1, Wg2, bg2d, *sels)


def kernel(query, episode_keys, episode_values, Wq, Wv, Wg, bg, top_k):
    del top_k
    keys_pad = jnp.pad(episode_keys, ((0, NPAD - N_EP), (0, 0)))
    cv, ci = _run_topk(query, Wq, keys_pad)
    cvt = cv.transpose(1, 0, 2).reshape(B, NCAND)
    cit = ci.transpose(1, 0, 2).reshape(B, NCAND)
    cvp = jnp.pad(cvt, ((0, 0), (0, CPAD - NCAND)), constant_values=NEG)
    cip = jnp.pad(cit, ((0, 0), (0, CPAD - NCAND)))
    tv, ti = _run_select(cvp, cip)
    idx2d = ti.reshape(_NTOT // _CH, _CH)
    sel = _make_sc_gather()(episode_values, idx2d)
    sel3 = sel.reshape(B, K, D)
    sels = [sel3[:, j, :] for j in range(K)]
    return _run_tail(query, tv, Wv, Wg[:, :D], Wg[:, D:], bg.reshape(1, D),
                     sels)


# speedup vs baseline: 1.8853x; 1.8853x over previous
import functools

import jax
import jax.numpy as jnp
from jax import lax
from jax.experimental import pallas as pl
from jax.experimental.pallas import tpu as pltpu
from jax.experimental.pallas import tpu_sc as plsc

B = 1024
D = 128
N_EP = 100000
K = 8
CBLK = 2048
NBLK = (N_EP + CBLK - 1) // CBLK
NPAD = NBLK * CBLK
NCAND = NBLK * K
CPAD = 512
NEG = -1e30
BIGF = 1e9


def _topk_body(query_ref, wq_ref, keys_ref, cv_ref, ci_ref, qn_ref, thr_ref):
    i = pl.program_id(0)

    @pl.when(i == 0)
    def _init():
        q = lax.dot_general(query_ref[...], wq_ref[...],
                            (((1,), (1,)), ((), ())),
                            preferred_element_type=jnp.float32)
        n = jnp.sqrt(jnp.sum(q * q, axis=1, keepdims=True))
        qn_ref[...] = q / jnp.maximum(n, 1e-12)
        thr_ref[...] = jnp.full((B,), NEG, jnp.float32)

    keys = keys_ref[...]
    kn = jnp.sqrt(jnp.sum(keys * keys, axis=1, keepdims=True))
    keysn = keys / jnp.maximum(kn, 1e-12)
    sim = lax.dot_general(qn_ref[...], keysn,
                          (((1,), (1,)), ((), ())),
                          preferred_element_type=jnp.float32)
    base = i * CBLK
    limitf = (N_EP - base).astype(jnp.float32)
    rowi = lax.broadcasted_iota(jnp.int32, (1, CBLK), 1).astype(jnp.float32)
    bias = jnp.where(rowi < limitf, 0.0, NEG)
    sim = sim + bias
    lcolf = lax.broadcasted_iota(jnp.int32, (B, CBLK), 1).astype(jnp.float32)
    basef = base.astype(jnp.float32)

    thr = thr_ref[...]
    cnt = jnp.sum((sim >= thr[:, None]).astype(jnp.float32), axis=1)
    need = jnp.max(cnt)

    H = K // 2
    cand_v, cand_i = [], []
    for t in range(H):
        m = jnp.max(sim, axis=1)
        lif = jnp.min(jnp.where(sim == m[:, None], lcolf, BIGF), axis=1)
        cand_v.append(m[:, None])
        cand_i.append(lif[:, None] + basef)
        sim = jnp.where(lcolf == lif[:, None], NEG, sim)
    cv_ref[0, :, 0:H] = jnp.concatenate(cand_v, axis=1)
    ci_ref[0, :, 0:H] = jnp.concatenate(cand_i, axis=1)
    cv_ref[0, :, H:K] = jnp.full((B, K - H), NEG, jnp.float32)
    ci_ref[0, :, H:K] = jnp.zeros((B, K - H), jnp.float32)

    @pl.when(need > float(H))
    def _ext_tail():
        s = sim
        tv, ti = [], []
        for t in range(K - H):
            m = jnp.max(s, axis=1)
            lif = jnp.min(jnp.where(s == m[:, None], lcolf, BIGF), axis=1)
            tv.append(m[:, None])
            ti.append(lif[:, None] + basef)
            if t < K - H - 1:
                s = jnp.where(lcolf == lif[:, None], NEG, s)
        cv_ref[0, :, H:K] = jnp.concatenate(tv, axis=1)
        ci_ref[0, :, H:K] = jnp.concatenate(ti, axis=1)
        thr_ref[...] = jnp.maximum(thr, tv[-1][:, 0])


def _run_topk(query, Wq, keys_pad):
    return pl.pallas_call(
        _topk_body,
        grid=(NBLK,),
        in_specs=[
            pl.BlockSpec((B, D), lambda i: (0, 0)),
            pl.BlockSpec((D, D), lambda i: (0, 0)),
            pl.BlockSpec((CBLK, D), lambda i: (i, 0)),
        ],
        out_specs=[
            pl.BlockSpec((1, B, K), lambda i: (i, 0, 0)),
            pl.BlockSpec((1, B, K), lambda i: (i, 0, 0)),
        ],
        out_shape=[
            jax.ShapeDtypeStruct((NBLK, B, K), jnp.float32),
            jax.ShapeDtypeStruct((NBLK, B, K), jnp.float32),
        ],
        scratch_shapes=[pltpu.VMEM((B, D), jnp.float32),
                        pltpu.VMEM((B,), jnp.float32)],
    )(query, Wq, keys_pad)


def _select_body(cv_ref, ci_ref, tv_ref, ti_ref):
    v = cv_ref[...]
    ci = ci_ref[...]
    pos = lax.broadcasted_iota(jnp.int32, (B, CPAD), 1).astype(jnp.float32)
    new_v, new_i = [], []
    for _ in range(K):
        m = jnp.max(v, axis=1)
        p = jnp.min(jnp.where(v == m[:, None], pos, BIGF), axis=1)
        hit = pos == p[:, None]
        new_v.append(m[:, None])
        new_i.append(jnp.max(jnp.where(hit, ci, -1.0), axis=1)[:, None])
        v = jnp.where(hit, NEG, v)
    tv_ref[...] = jnp.concatenate(new_v, axis=1)
    ti_ref[...] = jnp.concatenate(new_i, axis=1).astype(jnp.int32)


def _run_select(cvp, cip):
    return pl.pallas_call(
        _select_body,
        out_shape=[
            jax.ShapeDtypeStruct((B, K), jnp.float32),
            jax.ShapeDtypeStruct((B, K), jnp.int32),
        ],
    )(cvp, cip)


_NTOT = B * K
_CH = 128


def _make_sc_gather():
    info = plsc.get_sparse_core_info()
    nc, ns = info.num_cores, info.num_subcores
    nw = nc * ns
    bpw = _NTOT // nw
    nch = bpw // _CH
    mesh = plsc.VectorSubcoreMesh(core_axis_name="c", subcore_axis_name="s")

    @functools.partial(
        pl.kernel, mesh=mesh,
        out_type=jax.ShapeDtypeStruct((_NTOT, D), jnp.float32),
        scratch_types=[
            pltpu.VMEM((nch, _CH), jnp.int32),
            pltpu.VMEM((bpw, D), jnp.float32),
            pltpu.SemaphoreType.DMA,
        ],
    )
    def gather_k(values_hbm, idx_hbm, out_hbm, idx_v, rows_v, sem):
        wid = lax.axis_index("s") * nc + lax.axis_index("c")
        pltpu.sync_copy(idx_hbm.at[pl.ds(wid * nch, nch)], idx_v)
        cps = [
            pltpu.async_copy(values_hbm.at[idx_v.at[j]],
                             rows_v.at[pl.ds(j * _CH, _CH)], sem)
            for j in range(nch)
        ]
        for cp in cps:
            cp.wait()
        pltpu.sync_copy(rows_v, out_hbm.at[pl.ds(wid * bpw, bpw)])

    return gather_k


def _tail_body(query_ref, tv_ref, wv_ref, wg1_ref, wg2_ref, bg_ref,
               *sel_and_out):
    sel_refs = sel_and_out[:K]
    out_ref = sel_and_out[K]
    z = tv_ref[...] * 5.0
    m = jnp.max(z, axis=1, keepdims=True)
    e = jnp.exp(z - m)
    w = e / jnp.sum(e, axis=1, keepdims=True)
    r = w[:, 0:1] * sel_refs[0][...]
    for j in range(1, K):
        r = r + w[:, j:j + 1] * sel_refs[j][...]
    proj = lax.dot_general(r, wv_ref[...], (((1,), (1,)), ((), ())),
                           preferred_element_type=jnp.float32)
    g = (lax.dot_general(query_ref[...], wg1_ref[...],
                         (((1,), (1,)), ((), ())),
                         preferred_element_type=jnp.float32)
         + lax.dot_general(proj, wg2_ref[...], (((1,), (1,)), ((), ())),
                           preferred_element_type=jnp.float32)
         + bg_ref[...])
    gate = jax.nn.sigmoid(g)
    out_ref[...] = gate * proj


def _run_tail(query, tv, Wv, Wg1, Wg2, bg2d, sels):
    return pl.pallas_call(
        _tail_body,
        out_shape=jax.ShapeDtypeStruct((B, D), jnp.float32),
    )(query, tv, Wv, Wg1, Wg2, bg2d, *sels)


def kernel(query, episode_keys, episode_values, Wq, Wv, Wg, bg, top_k):
    del top_k
    keys_pad = jnp.pad(episode_keys, ((0, NPAD - N_EP), (0, 0)))
    cv, ci = _run_topk(query, Wq, keys_pad)
    cvt = cv.transpose(1, 0, 2).reshape(B, NCAND)
    cit = ci.transpose(1, 0, 2).reshape(B, NCAND)
    cvp = jnp.pad(cvt, ((0, 0), (0, CPAD - NCAND)), constant_values=NEG)
    cip = jnp.pad(cit, ((0, 0), (0, CPAD - NCAND)))
    tv, ti = _run_select(cvp, cip)
    idx2d = ti.reshape(_NTOT // _CH, _CH)
    sel = _make_sc_gather()(episode_values, idx2d)
    sel3 = sel.reshape(B, K, D)
    sels = [sel3[:, j, :] for j in range(K)]
    return _run_tail(query, tv, Wv, Wg[:, :D], Wg[:, D:], bg.reshape(1, D),
                     sels)


# speedup vs baseline: 2.0247x; 1.0739x over previous
import functools

import jax
import jax.numpy as jnp
from jax import lax
from jax.experimental import pallas as pl
from jax.experimental.pallas import tpu as pltpu
from jax.experimental.pallas import tpu_sc as plsc

B = 1024
D = 128
N_EP = 100000
K = 8
CBLK = 2000
NBLK = N_EP // CBLK
NCAND = NBLK * K
NEG = -1e30
BIGF = 1e9


def _topk_body(query_ref, wq_ref, keys_ref, cv_ref, ci_ref, qn_ref):
    i = pl.program_id(0)

    @pl.when(i == 0)
    def _init():
        q = lax.dot_general(query_ref[...], wq_ref[...],
                            (((1,), (1,)), ((), ())),
                            preferred_element_type=jnp.float32)
        n = jnp.sqrt(jnp.sum(q * q, axis=1, keepdims=True))
        qn_ref[...] = q / jnp.maximum(n, 1e-12)

    keys = keys_ref[...]
    kn = jnp.sqrt(jnp.sum(keys * keys, axis=1, keepdims=True))
    keysn = keys / jnp.maximum(kn, 1e-12)
    sim = lax.dot_general(qn_ref[...], keysn,
                          (((1,), (1,)), ((), ())),
                          preferred_element_type=jnp.float32)
    lcolf = lax.broadcasted_iota(jnp.int32, (B, CBLK), 1).astype(jnp.float32)
    basef = (i * CBLK).astype(jnp.float32)

    cand_v, cand_i = [], []
    for t in range(K):
        m = jnp.max(sim, axis=1)
        lif = jnp.min(jnp.where(sim == m[:, None], lcolf, BIGF), axis=1)
        cand_v.append(m[:, None])
        cand_i.append(lif[:, None] + basef)
        if t < K - 1:
            sim = jnp.where(lcolf == lif[:, None], NEG, sim)
    cv_ref[0] = jnp.concatenate(cand_v, axis=1)
    ci_ref[0] = jnp.concatenate(cand_i, axis=1)


def _run_topk(query, Wq, keys):
    return pl.pallas_call(
        _topk_body,
        grid=(NBLK,),
        in_specs=[
            pl.BlockSpec((B, D), lambda i: (0, 0)),
            pl.BlockSpec((D, D), lambda i: (0, 0)),
            pl.BlockSpec((CBLK, D), lambda i: (i, 0)),
        ],
        out_specs=[
            pl.BlockSpec((1, B, K), lambda i: (i, 0, 0)),
            pl.BlockSpec((1, B, K), lambda i: (i, 0, 0)),
        ],
        out_shape=[
            jax.ShapeDtypeStruct((NBLK, B, K), jnp.float32),
            jax.ShapeDtypeStruct((NBLK, B, K), jnp.float32),
        ],
        scratch_shapes=[pltpu.VMEM((B, D), jnp.float32)],
    )(query, Wq, keys)


def _select_body(cv_ref, ci_ref, tv_ref, ti_ref):
    v = cv_ref[...]
    ci = ci_ref[...]
    pos = lax.broadcasted_iota(jnp.int32, (B, NCAND), 1).astype(jnp.float32)
    new_v, new_i = [], []
    for _ in range(K):
        m = jnp.max(v, axis=1)
        p = jnp.min(jnp.where(v == m[:, None], pos, BIGF), axis=1)
        hit = pos == p[:, None]
        new_v.append(m[:, None])
        new_i.append(jnp.max(jnp.where(hit, ci, -1.0), axis=1)[:, None])
        v = jnp.where(hit, NEG, v)
    tv_ref[...] = jnp.concatenate(new_v, axis=1)
    ti_ref[...] = jnp.concatenate(new_i, axis=1).astype(jnp.int32)


def _run_select(cvt, cit):
    return pl.pallas_call(
        _select_body,
        out_shape=[
            jax.ShapeDtypeStruct((B, K), jnp.float32),
            jax.ShapeDtypeStruct((B, K), jnp.int32),
        ],
    )(cvt, cit)


_NTOT = B * K
_CH = 128


def _make_sc_gather():
    info = plsc.get_sparse_core_info()
    nc, ns = info.num_cores, info.num_subcores
    nw = nc * ns
    bpw = _NTOT // nw
    nch = bpw // _CH
    mesh = plsc.VectorSubcoreMesh(core_axis_name="c", subcore_axis_name="s")

    @functools.partial(
        pl.kernel, mesh=mesh,
        out_type=jax.ShapeDtypeStruct((_NTOT, D), jnp.float32),
        scratch_types=[
            pltpu.VMEM((nch, _CH), jnp.int32),
            pltpu.VMEM((bpw, D), jnp.float32),
            pltpu.SemaphoreType.DMA,
        ],
    )
    def gather_k(values_hbm, idx_hbm, out_hbm, idx_v, rows_v, sem):
        wid = lax.axis_index("s") * nc + lax.axis_index("c")
        pltpu.sync_copy(idx_hbm.at[pl.ds(wid * nch, nch)], idx_v)
        cps = [
            pltpu.async_copy(values_hbm.at[idx_v.at[j]],
                             rows_v.at[pl.ds(j * _CH, _CH)], sem)
            for j in range(nch)
        ]
        for cp in cps:
            cp.wait()
        pltpu.sync_copy(rows_v, out_hbm.at[pl.ds(wid * bpw, bpw)])

    return gather_k


def _tail_body(query_ref, tv_ref, wv_ref, wg1_ref, wg2_ref, bg_ref, sel_ref,
               out_ref):
    z = tv_ref[...] * 5.0
    m = jnp.max(z, axis=1, keepdims=True)
    e = jnp.exp(z - m)
    w = e / jnp.sum(e, axis=1, keepdims=True)
    r = w[:, 0:1] * sel_ref[:, 0:D]
    for j in range(1, K):
        r = r + w[:, j:j + 1] * sel_ref[:, j * D:(j + 1) * D]
    proj = lax.dot_general(r, wv_ref[...], (((1,), (1,)), ((), ())),
                           preferred_element_type=jnp.float32)
    g = (lax.dot_general(query_ref[...], wg1_ref[...],
                         (((1,), (1,)), ((), ())),
                         preferred_element_type=jnp.float32)
         + lax.dot_general(proj, wg2_ref[...], (((1,), (1,)), ((), ())),
                           preferred_element_type=jnp.float32)
         + bg_ref[...])
    gate = jax.nn.sigmoid(g)
    out_ref[...] = gate * proj


def _run_tail(query, tv, Wv, Wg1, Wg2, bg2d, sel2d):
    return pl.pallas_call(
        _tail_body,
        out_shape=jax.ShapeDtypeStruct((B, D), jnp.float32),
    )(query, tv, Wv, Wg1, Wg2, bg2d, sel2d)


def kernel(query, episode_keys, episode_values, Wq, Wv, Wg, bg, top_k):
    del top_k
    cv, ci = _run_topk(query, Wq, episode_keys)
    cvt = cv.transpose(1, 0, 2).reshape(B, NCAND)
    cit = ci.transpose(1, 0, 2).reshape(B, NCAND)
    tv, ti = _run_select(cvt, cit)
    idx2d = ti.reshape(_NTOT // _CH, _CH)
    sel = _make_sc_gather()(episode_values, idx2d)
    sel2d = sel.reshape(B, K * D)
    return _run_tail(query, tv, Wv, Wg[:, :D], Wg[:, D:], bg.reshape(1, D),
                     sel2d)


# speedup vs baseline: 2.2642x; 1.1183x over previous
import functools

import jax
import jax.numpy as jnp
from jax import lax
from jax.experimental import pallas as pl
from jax.experimental.pallas import tpu as pltpu
from jax.experimental.pallas import tpu_sc as plsc

B = 1024
D = 128
N_EP = 100000
K = 8
CBLK = 4000
NBLK = N_EP // CBLK
NCAND = NBLK * K
NEG = -1e30
BIGF = 1e9


def _topk_body(query_ref, wq_ref, keys_ref, cv_ref, ci_ref, qn_ref):
    i = pl.program_id(0)

    @pl.when(i == 0)
    def _init():
        q = lax.dot_general(query_ref[...], wq_ref[...],
                            (((1,), (1,)), ((), ())),
                            preferred_element_type=jnp.float32)
        n = jnp.sqrt(jnp.sum(q * q, axis=1, keepdims=True))
        qn_ref[...] = q / jnp.maximum(n, 1e-12)

    keys = keys_ref[...]
    kn = jnp.sqrt(jnp.sum(keys * keys, axis=1, keepdims=True))
    keysn = keys / jnp.maximum(kn, 1e-12)
    sim = lax.dot_general(qn_ref[...], keysn,
                          (((1,), (1,)), ((), ())),
                          preferred_element_type=jnp.float32)
    lcolf = lax.broadcasted_iota(jnp.int32, (B, CBLK), 1).astype(jnp.float32)
    basef = (i * CBLK).astype(jnp.float32)

    cand_v, cand_i = [], []
    for t in range(K):
        m = jnp.max(sim, axis=1)
        lif = jnp.min(jnp.where(sim == m[:, None], lcolf, BIGF), axis=1)
        cand_v.append(m[:, None])
        cand_i.append(lif[:, None] + basef)
        if t < K - 1:
            sim = jnp.where(lcolf == lif[:, None], NEG, sim)
    cv_ref[0] = jnp.concatenate(cand_v, axis=1)
    ci_ref[0] = jnp.concatenate(cand_i, axis=1)


def _run_topk(query, Wq, keys):
    return pl.pallas_call(
        _topk_body,
        grid=(NBLK,),
        in_specs=[
            pl.BlockSpec((B, D), lambda i: (0, 0)),
            pl.BlockSpec((D, D), lambda i: (0, 0)),
            pl.BlockSpec((CBLK, D), lambda i: (i, 0)),
        ],
        out_specs=[
            pl.BlockSpec((1, B, K), lambda i: (i, 0, 0)),
            pl.BlockSpec((1, B, K), lambda i: (i, 0, 0)),
        ],
        out_shape=[
            jax.ShapeDtypeStruct((NBLK, B, K), jnp.float32),
            jax.ShapeDtypeStruct((NBLK, B, K), jnp.float32),
        ],
        scratch_shapes=[pltpu.VMEM((B, D), jnp.float32)],
    )(query, Wq, keys)


def _select_body(cv_ref, ci_ref, tv_ref, ti_ref):
    v = cv_ref[...]
    ci = ci_ref[...]
    pos = lax.broadcasted_iota(jnp.int32, (B, NCAND), 1).astype(jnp.float32)
    new_v, new_i = [], []
    for _ in range(K):
        m = jnp.max(v, axis=1)
        p = jnp.min(jnp.where(v == m[:, None], pos, BIGF), axis=1)
        hit = pos == p[:, None]
        new_v.append(m[:, None])
        new_i.append(jnp.max(jnp.where(hit, ci, -1.0), axis=1)[:, None])
        v = jnp.where(hit, NEG, v)
    tv_ref[...] = jnp.concatenate(new_v, axis=1)
    ti_ref[...] = jnp.concatenate(new_i, axis=1).astype(jnp.int32)


def _run_select(cvt, cit):
    return pl.pallas_call(
        _select_body,
        out_shape=[
            jax.ShapeDtypeStruct((B, K), jnp.float32),
            jax.ShapeDtypeStruct((B, K), jnp.int32),
        ],
    )(cvt, cit)


_NTOT = B * K
_CH = 128


def _make_sc_gather():
    info = plsc.get_sparse_core_info()
    nc, ns = info.num_cores, info.num_subcores
    nw = nc * ns
    bpw = _NTOT // nw
    nch = bpw // _CH
    mesh = plsc.VectorSubcoreMesh(core_axis_name="c", subcore_axis_name="s")

    @functools.partial(
        pl.kernel, mesh=mesh,
        out_type=jax.ShapeDtypeStruct((_NTOT, D), jnp.float32),
        scratch_types=[
            pltpu.VMEM((nch, _CH), jnp.int32),
            pltpu.VMEM((bpw, D), jnp.float32),
            pltpu.SemaphoreType.DMA,
        ],
    )
    def gather_k(values_hbm, idx_hbm, out_hbm, idx_v, rows_v, sem):
        wid = lax.axis_index("s") * nc + lax.axis_index("c")
        pltpu.sync_copy(idx_hbm.at[pl.ds(wid * nch, nch)], idx_v)
        cps = [
            pltpu.async_copy(values_hbm.at[idx_v.at[j]],
                             rows_v.at[pl.ds(j * _CH, _CH)], sem)
            for j in range(nch)
        ]
        for cp in cps:
            cp.wait()
        pltpu.sync_copy(rows_v, out_hbm.at[pl.ds(wid * bpw, bpw)])

    return gather_k


def _tail_body(query_ref, tv_ref, wv_ref, wg1_ref, wg2_ref, bg_ref, sel_ref,
               out_ref):
    z = tv_ref[...] * 5.0
    m = jnp.max(z, axis=1, keepdims=True)
    e = jnp.exp(z - m)
    w = e / jnp.sum(e, axis=1, keepdims=True)
    r = w[:, 0:1] * sel_ref[:, 0:D]
    for j in range(1, K):
        r = r + w[:, j:j + 1] * sel_ref[:, j * D:(j + 1) * D]
    proj = lax.dot_general(r, wv_ref[...], (((1,), (1,)), ((), ())),
                           preferred_element_type=jnp.float32)
    g = (lax.dot_general(query_ref[...], wg1_ref[...],
                         (((1,), (1,)), ((), ())),
                         preferred_element_type=jnp.float32)
         + lax.dot_general(proj, wg2_ref[...], (((1,), (1,)), ((), ())),
                           preferred_element_type=jnp.float32)
         + bg_ref[...])
    gate = jax.nn.sigmoid(g)
    out_ref[...] = gate * proj


def _run_tail(query, tv, Wv, Wg1, Wg2, bg2d, sel2d):
    return pl.pallas_call(
        _tail_body,
        out_shape=jax.ShapeDtypeStruct((B, D), jnp.float32),
    )(query, tv, Wv, Wg1, Wg2, bg2d, sel2d)


def kernel(query, episode_keys, episode_values, Wq, Wv, Wg, bg, top_k):
    del top_k
    cv, ci = _run_topk(query, Wq, episode_keys)
    cvt = cv.transpose(1, 0, 2).reshape(B, NCAND)
    cit = ci.transpose(1, 0, 2).reshape(B, NCAND)
    tv, ti = _run_select(cvt, cit)
    idx2d = ti.reshape(_NTOT // _CH, _CH)
    sel = _make_sc_gather()(episode_values, idx2d)
    sel2d = sel.reshape(B, K * D)
    return _run_tail(query, tv, Wv, Wg[:, :D], Wg[:, D:], bg.reshape(1, D),
                     sel2d)
